# Initial kernel scaffold; baseline (speedup 1.0000x reference)
import jax
import jax.numpy as jnp
from jax.experimental import pallas as pl


def kernel(input, weight):
    raise NotImplementedError("write your pallas kernel here")


# speedup vs baseline: 1.8289x; 1.8289x over previous
import functools

import jax
import jax.numpy as jnp
from jax import lax
from jax.experimental import pallas as pl
from jax.experimental.pallas import tpu as pltpu
from jax.experimental.pallas import tpu_sc as plsc

_EMBED_DIM = 64
_NC = 2
_NS = 16
_NW = _NC * _NS
_CHUNK = 128
_NBUF = 4


def _make_kernel(n_idx):
    assert n_idx % (_NW * _CHUNK) == 0
    per_w = n_idx // _NW
    chunks = per_w // _CHUNK
    assert chunks % _NBUF == 0 and chunks > _NBUF

    mesh = plsc.VectorSubcoreMesh(core_axis_name="c", subcore_axis_name="s")

    @functools.partial(
        pl.kernel,
        out_type=jax.ShapeDtypeStruct((_NW, chunks, _CHUNK, _EMBED_DIM), jnp.float32),
        mesh=mesh,
        scratch_types=[
            pltpu.VMEM((chunks, _CHUNK), jnp.int32),
            pltpu.VMEM((_NBUF, _CHUNK, _EMBED_DIM), jnp.float32),
            pltpu.SemaphoreType.DMA,
        ],
        compiler_params=pltpu.CompilerParams(use_tc_tiling_on_sc=False),
    )
    def emb(table_hbm, idx_hbm, out_hbm, idx_v, rows, sem):
        wid = lax.axis_index("s") * _NC + lax.axis_index("c")
        pltpu.sync_copy(idx_hbm.at[wid], idx_v)

        @pl.loop(0, chunks, step=_NBUF)
        def _(g0):
            descs = [
                pltpu.async_copy(table_hbm.at[idx_v.at[g0 + b]], rows.at[b], sem)
                for b in range(_NBUF)
            ]
            for d in descs:
                d.wait()
            pltpu.sync_copy(rows, out_hbm.at[wid, pl.ds(g0, _NBUF)])

    return emb, chunks


def kernel(input, weight):
    batch, seq = input.shape
    n_idx = batch * seq
    emb, chunks = _make_kernel(n_idx)
    idx = input.reshape(_NW, chunks, _CHUNK).astype(jnp.int32)
    out = emb(weight, idx)
    return out.reshape(batch, seq, _EMBED_DIM)


# speedup vs baseline: 1.8580x; 1.0159x over previous
import functools

import jax
import jax.numpy as jnp
from jax import lax
from jax.experimental import pallas as pl
from jax.experimental.pallas import tpu as pltpu
from jax.experimental.pallas import tpu_sc as plsc

_EMBED_DIM = 64
_NC = 2
_NS = 16
_NW = _NC * _NS
_CHUNK = 128
_NBUF = 8


def _make_kernel(n_idx):
    assert n_idx % (_NW * _CHUNK) == 0
    per_w = n_idx // _NW
    chunks = per_w // _CHUNK
    assert chunks % _NBUF == 0 and chunks > _NBUF

    mesh = plsc.VectorSubcoreMesh(core_axis_name="c", subcore_axis_name="s")

    @functools.partial(
        pl.kernel,
        out_type=jax.ShapeDtypeStruct((_NW, chunks, _CHUNK, _EMBED_DIM), jnp.float32),
        mesh=mesh,
        scratch_types=[
            pltpu.VMEM((chunks, _CHUNK), jnp.int32),
            pltpu.VMEM((_NBUF, _CHUNK, _EMBED_DIM), jnp.float32),
            pltpu.SemaphoreType.DMA,
        ],
        compiler_params=pltpu.CompilerParams(use_tc_tiling_on_sc=False),
    )
    def emb(table_hbm, idx_hbm, out_hbm, idx_v, rows, sem):
        wid = lax.axis_index("s") * _NC + lax.axis_index("c")
        pltpu.sync_copy(idx_hbm.at[wid], idx_v)

        @pl.loop(0, chunks, step=_NBUF)
        def _(g0):
            descs = [
                pltpu.async_copy(table_hbm.at[idx_v.at[g0 + b]], rows.at[b], sem)
                for b in range(_NBUF)
            ]
            for d in descs:
                d.wait()
            pltpu.sync_copy(rows, out_hbm.at[wid, pl.ds(g0, _NBUF)])

    return emb, chunks


def kernel(input, weight):
    batch, seq = input.shape
    n_idx = batch * seq
    emb, chunks = _make_kernel(n_idx)
    idx = input.reshape(_NW, chunks, _CHUNK).astype(jnp.int32)
    out = emb(weight, idx)
    return out.reshape(batch, seq, _EMBED_DIM)


# speedup vs baseline: 1.8668x; 1.0048x over previous
import functools

import jax
import jax.numpy as jnp
from jax import lax
from jax.experimental import pallas as pl
from jax.experimental.pallas import tpu as pltpu
from jax.experimental.pallas import tpu_sc as plsc

_EMBED_DIM = 64
_NC = 2
_NS = 16
_NW = _NC * _NS
_CHUNK = 128
_NBUF = 4


def _make_kernel(n_idx):
    assert n_idx % (_NW * _CHUNK) == 0
    per_w = n_idx // _NW
    chunks = per_w // _CHUNK
    groups = chunks // _NBUF
    assert chunks % _NBUF == 0 and groups % 2 == 0 and groups >= 4

    mesh = plsc.VectorSubcoreMesh(core_axis_name="c", subcore_axis_name="s")

    @functools.partial(
        pl.kernel,
        out_type=jax.ShapeDtypeStruct((_NW, chunks, _CHUNK, _EMBED_DIM), jnp.float32),
        mesh=mesh,
        scratch_types=[
            pltpu.VMEM((chunks, _CHUNK), jnp.int32),
            pltpu.VMEM((_NBUF, _CHUNK, _EMBED_DIM), jnp.float32),
            pltpu.VMEM((_NBUF, _CHUNK, _EMBED_DIM), jnp.float32),
            pltpu.SemaphoreType.DMA,
            pltpu.SemaphoreType.DMA,
            pltpu.SemaphoreType.DMA,
            pltpu.SemaphoreType.DMA,
        ],
        compiler_params=pltpu.CompilerParams(use_tc_tiling_on_sc=False),
    )
    def emb(table_hbm, idx_hbm, out_hbm, idx_v, buf_a, buf_b, gsem_a, gsem_b, wsem_a, wsem_b):
        wid = lax.axis_index("s") * _NC + lax.axis_index("c")
        pltpu.sync_copy(idx_hbm.at[wid], idx_v)

        def fire(buf, gsem, g):
            return [
                pltpu.async_copy(
                    table_hbm.at[idx_v.at[g * _NBUF + b]], buf.at[b], gsem
                )
                for b in range(_NBUF)
            ]

        def write(buf, wsem, g):
            pltpu.async_copy(buf, out_hbm.at[wid, pl.ds(g * _NBUF, _NBUF)], wsem)

        def wait_write(buf, wsem, g):
            pltpu.make_async_copy(
                buf, out_hbm.at[wid, pl.ds(g * _NBUF, _NBUF)], wsem
            ).wait()

        def drain_fire_write(buf, gsem, wsem, descs, g):
            for d in descs:
                d.wait()
            write(buf, wsem, g)

        descs_a = fire(buf_a, gsem_a, 0)
        descs_b = fire(buf_b, gsem_b, 1)
        drain_fire_write(buf_a, gsem_a, wsem_a, descs_a, 0)
        drain_fire_write(buf_b, gsem_b, wsem_b, descs_b, 1)

        @pl.loop(2, groups, step=2)
        def _(ga):
            wait_write(buf_a, wsem_a, ga - 2)
            descs_a = fire(buf_a, gsem_a, ga)
            wait_write(buf_b, wsem_b, ga - 1)
            descs_b = fire(buf_b, gsem_b, ga + 1)
            drain_fire_write(buf_a, gsem_a, wsem_a, descs_a, ga)
            drain_fire_write(buf_b, gsem_b, wsem_b, descs_b, ga + 1)

        wait_write(buf_a, wsem_a, groups - 2)
        wait_write(buf_b, wsem_b, groups - 1)

    return emb, chunks


def kernel(input, weight):
    batch, seq = input.shape
    n_idx = batch * seq
    emb, chunks = _make_kernel(n_idx)
    idx = input.reshape(_NW, chunks, _CHUNK).astype(jnp.int32)
    out = emb(weight, idx)
    return out.reshape(batch, seq, _EMBED_DIM)
